# Initial kernel scaffold; baseline (speedup 1.0000x reference)
import jax
import jax.numpy as jnp
from jax.experimental import pallas as pl


def kernel(event_type_ids, proc_path_ids, tgt_path_ids, signing_ids, numerical, temporal, event_table, proc_path_table, tgt_path_table, signing_table, num_W, num_b, temp_W, temp_b, proj_W, proj_b, ln_gamma, ln_beta, pe):
    raise NotImplementedError("write your pallas kernel here")


# speedup vs baseline: 8.2804x; 8.2804x over previous
import functools
import math

import jax
import jax.numpy as jnp
from jax import lax
from jax.experimental import pallas as pl
from jax.experimental.pallas import tpu as pltpu
from jax.experimental.pallas import tpu_sc as plsc

_C = 128


def _tc_body(e_ref, p_ref, t_ref, s_ref, pid_ref, tid_ref, nt_ref,
             w_ref, wnt_ref, bias_ref, g_ref, b_ref, pe_ref, o_ref):
    R, S, E = e_ref.shape
    M = R * S
    e = e_ref[...].reshape(M, E)
    p = p_ref[...].reshape(M, p_ref.shape[2])
    t = t_ref[...].reshape(M, t_ref.shape[2])
    s = s_ref[...].reshape(M, s_ref.shape[2])
    pidf = (pid_ref[...].reshape(M, pid_ref.shape[2]) != 0).astype(jnp.float32)
    tidf = (tid_ref[...].reshape(M, tid_ref.shape[2]) != 0).astype(jnp.float32)
    pcnt = jnp.maximum(jnp.sum(pidf, axis=1, keepdims=True), 1.0)
    tcnt = jnp.maximum(jnp.sum(tidf, axis=1, keepdims=True), 1.0)
    p = p / pcnt
    t = t / tcnt
    cat = jnp.concatenate([e, p, t, s], axis=1)
    x = jnp.dot(cat, w_ref[...], preferred_element_type=jnp.float32)
    x = x + jnp.dot(nt_ref[...].reshape(M, nt_ref.shape[2]), wnt_ref[...],
                    preferred_element_type=jnp.float32)
    x = x + bias_ref[...]
    mu = jnp.mean(x, axis=1, keepdims=True)
    xc = x - mu
    var = jnp.mean(xc * xc, axis=1, keepdims=True)
    y = xc * lax.rsqrt(var + 1e-5) * g_ref[...] + b_ref[...]
    y = 0.5 * y * (1.0 + lax.erf(y * (1.0 / math.sqrt(2.0))))
    o_ref[...] = y.reshape(R, S, y.shape[1]) + pe_ref[...][None]


def kernel(event_type_ids, proc_path_ids, tgt_path_ids, signing_ids,
           numerical, temporal, event_table, proc_path_table, tgt_path_table,
           signing_table, num_W, num_b, temp_W, temp_b, proj_W, proj_b,
           ln_gamma, ln_beta, pe):
    B, S, P = proc_path_ids.shape
    BT = B * S
    E = event_table.shape[1]
    D = proc_path_table.shape[1]
    SG = signing_table.shape[1]
    DM = proj_W.shape[1]
    f32 = jnp.float32

    info = plsc.get_sparse_core_info()
    NC, NS = info.num_cores, info.num_subcores
    NW = NC * NS
    tokw = BT // NW
    C = _C
    nchunk = tokw // C
    IR = (C * P) // 128

    pids2d = proc_path_ids.reshape(-1, 128)
    tids2d = tgt_path_ids.reshape(-1, 128)
    eids = event_type_ids.reshape(-1)
    sids = signing_ids.reshape(-1)

    mesh = plsc.VectorSubcoreMesh(core_axis_name="c", subcore_axis_name="s")

    @functools.partial(
        pl.kernel,
        out_type=(jax.ShapeDtypeStruct((BT, E), f32),
                  jax.ShapeDtypeStruct((BT, D), f32),
                  jax.ShapeDtypeStruct((BT, D), f32),
                  jax.ShapeDtypeStruct((BT, SG), f32)),
        mesh=mesh,
        scratch_types=[
            pltpu.VMEM((IR, 128), jnp.int32),
            pltpu.VMEM((IR, 128), jnp.int32),
            pltpu.VMEM((C,), jnp.int32),
            pltpu.VMEM((C,), jnp.int32),
            pltpu.VMEM((C * P // 2, D), f32),
            pltpu.VMEM((C * P // 2, D), f32),
            pltpu.VMEM((C, E), f32),
            pltpu.VMEM((C, SG), f32),
            pltpu.VMEM((C, D), f32),
            pltpu.VMEM((C, D), f32),
            pltpu.SemaphoreType.DMA,
        ],
        compiler_params=pltpu.CompilerParams(use_tc_tiling_on_sc=False),
    )
    def sc_pool(pids_h, tids_h, eids_h, sids_h, ptab_h, ttab_h, etab_h, stab_h,
                e_out, p_out, t_out, s_out,
                pidx, tidx, eidx, sidx, prow, trow, erow, srow, psum, tsum,
                sem):
        wid = lax.axis_index("s") * NC + lax.axis_index("c")
        tok_base = wid * tokw

        @pl.loop(0, nchunk)
        def _chunk(i):
            tok0 = pl.multiple_of(tok_base + i * C, C)
            irow0 = pl.multiple_of(tok0 * P // 128, 8)
            pltpu.sync_copy(pids_h.at[pl.ds(irow0, IR), :], pidx)
            pltpu.sync_copy(tids_h.at[pl.ds(irow0, IR), :], tidx)
            pltpu.sync_copy(eids_h.at[pl.ds(tok0, C)], eidx)
            pltpu.sync_copy(sids_h.at[pl.ds(tok0, C)], sidx)
            cps = [pltpu.async_copy(etab_h.at[eidx], erow, sem),
                   pltpu.async_copy(stab_h.at[sidx], srow, sem)]
            for cp in cps:
                cp.wait()
            pltpu.sync_copy(erow, e_out.at[pl.ds(tok0, C), :])
            pltpu.sync_copy(srow, s_out.at[pl.ds(tok0, C), :])

            for w in range(2):
                gps = []
                for j in range(IR // 2):
                    jj = w * (IR // 2) + j
                    gps.append(pltpu.async_copy(
                        ptab_h.at[pidx.at[jj]], prow.at[pl.ds(j * 128, 128)],
                        sem))
                    gps.append(pltpu.async_copy(
                        ttab_h.at[tidx.at[jj]], trow.at[pl.ds(j * 128, 128)],
                        sem))
                for cp in gps:
                    cp.wait()

                @pl.loop(0, C // 2)
                def _tok(t):
                    base = t * P
                    tt = w * (C // 2) + t
                    for c4 in range(D // 16):
                        sl = pl.ds(c4 * 16, 16)
                        ap = prow[base, sl]
                        at = trow[base, sl]
                        for r in range(1, P):
                            ap = ap + prow[base + r, sl]
                            at = at + trow[base + r, sl]
                        psum[tt, sl] = ap
                        tsum[tt, sl] = at

            pltpu.sync_copy(psum, p_out.at[pl.ds(tok0, C), :])
            pltpu.sync_copy(tsum, t_out.at[pl.ds(tok0, C), :])

    e_emb, p_sum, t_sum, s_emb = sc_pool(
        pids2d, tids2d, eids, sids,
        proc_path_table, tgt_path_table, event_table, signing_table)

    w_n = proj_W[E + 2 * D + SG:E + 2 * D + SG + num_W.shape[1]]
    w_t = proj_W[E + 2 * D + SG + num_W.shape[1]:]
    wnt = jnp.concatenate([num_W @ w_n, temp_W @ w_t], axis=0)
    bias2 = (proj_b + num_b @ w_n + temp_b @ w_t)[None, :]
    wcat = proj_W[:E + 2 * D + SG]
    nt = jnp.concatenate([numerical, temporal], axis=-1)
    pe_s = pe[0, :S, :]

    R = 16
    e3 = e_emb.reshape(B, S, E)
    p3 = p_sum.reshape(B, S, D)
    t3 = t_sum.reshape(B, S, D)
    s3 = s_emb.reshape(B, S, SG)
    NT = nt.shape[2]

    out = pl.pallas_call(
        _tc_body,
        grid=(B // R,),
        in_specs=[
            pl.BlockSpec((R, S, E), lambda i: (i, 0, 0)),
            pl.BlockSpec((R, S, D), lambda i: (i, 0, 0)),
            pl.BlockSpec((R, S, D), lambda i: (i, 0, 0)),
            pl.BlockSpec((R, S, SG), lambda i: (i, 0, 0)),
            pl.BlockSpec((R, S, P), lambda i: (i, 0, 0)),
            pl.BlockSpec((R, S, P), lambda i: (i, 0, 0)),
            pl.BlockSpec((R, S, NT), lambda i: (i, 0, 0)),
            pl.BlockSpec((E + 2 * D + SG, DM), lambda i: (0, 0)),
            pl.BlockSpec((NT, DM), lambda i: (0, 0)),
            pl.BlockSpec((1, DM), lambda i: (0, 0)),
            pl.BlockSpec((1, DM), lambda i: (0, 0)),
            pl.BlockSpec((1, DM), lambda i: (0, 0)),
            pl.BlockSpec((S, DM), lambda i: (0, 0)),
        ],
        out_specs=pl.BlockSpec((R, S, DM), lambda i: (i, 0, 0)),
        out_shape=jax.ShapeDtypeStruct((B, S, DM), f32),
    )(e3, p3, t3, s3, proc_path_ids, tgt_path_ids, nt,
      wcat, wnt, bias2, ln_gamma[None, :], ln_beta[None, :], pe_s)
    return out
